# baseline (speedup 1.0000x reference)
import jax
import jax.numpy as jnp
from jax import lax
from jax.experimental import pallas as pl
from jax.experimental.pallas import tpu as pltpu
from jax.experimental.pallas import tpu_sc as plsc

N = 32768
H = 128
NF = 4096
MIN_FRAG = 3.0

_R1 = 4096
_R1VM = 4096


def _proj(x, lw, lb, W, b):
    m = jnp.mean(x, axis=1, keepdims=True)
    xc = x - m
    v = jnp.mean(xc * xc, axis=1, keepdims=True)
    h = xc * lax.rsqrt(v + 1e-5) * lw + lb
    y = lax.dot_general(h, W, (((1,), (1,)), ((), ())),
                        preferred_element_type=jnp.float32) + b
    return y * jax.nn.sigmoid(y)


def _s1a_body(ss, lnsw, lnsb, Ws, bs, xa):
    ys = _proj(ss[...], lnsw[...], lnsb[...], Ws[...], bs[...])
    rs = jnp.sqrt(jnp.sum(ys * ys, axis=1, keepdims=True))
    fn = ys / jnp.maximum(rs, 1e-12)
    xa[...] = jnp.concatenate([fn, ys], axis=1)


def _s1yl_body(sl, lnlw, lnlb, Wl, bl, xb):
    xb[...] = _proj(sl[...], lnlw[...], lnlb[...], Wl[...], bl[...])


def _s1vm_body(v0, v1, v2, xb):
    a = v0[...]
    b = v1[...]
    c = v2[...]
    vm = ((jnp.sqrt(jnp.sum(a * a, axis=1)) + jnp.sqrt(jnp.sum(b * b, axis=1))
           + jnp.sqrt(jnp.sum(c * c, axis=1))) * (1.0 / 3.0))[:, None]
    vmn = vm / jnp.maximum(vm, 1e-12)
    ones = jnp.ones_like(vm)
    zeros = jnp.zeros((vm.shape[0], H - 3), jnp.float32)
    xb[...] = jnp.concatenate([ones, vm, vmn, zeros], axis=1)


_ROW = lambda i: (i, 0)
_FULL = lambda i: (0, 0)


def _stage1a(ss, lnsw, lnsb, Ws, bs):
    return pl.pallas_call(
        _s1a_body,
        grid=(N // _R1,),
        in_specs=[
            pl.BlockSpec((_R1, H), _ROW),
            pl.BlockSpec((1, H), _FULL),
            pl.BlockSpec((1, H), _FULL),
            pl.BlockSpec((H, H), _FULL),
            pl.BlockSpec((1, H), _FULL),
        ],
        out_specs=pl.BlockSpec((_R1, 2 * H), _ROW),
        out_shape=jax.ShapeDtypeStruct((N, 2 * H), jnp.float32),
    )(ss, lnsw, lnsb, Ws, bs)


def _stage1yl(sl, lnlw, lnlb, Wl, bl):
    return pl.pallas_call(
        _s1yl_body,
        grid=(N // _R1,),
        in_specs=[
            pl.BlockSpec((_R1, H), _ROW),
            pl.BlockSpec((1, H), _FULL),
            pl.BlockSpec((1, H), _FULL),
            pl.BlockSpec((H, H), _FULL),
            pl.BlockSpec((1, H), _FULL),
        ],
        out_specs=pl.BlockSpec((_R1, H), _ROW),
        out_shape=jax.ShapeDtypeStruct((N, H), jnp.float32),
    )(sl, lnlw, lnlb, Wl, bl)


def _stage1vm(vs):
    nb = N // _R1VM
    vr = jnp.reshape(vs, (3 * N, H))
    return pl.pallas_call(
        _s1vm_body,
        grid=(nb,),
        in_specs=[
            pl.BlockSpec((_R1VM, H), lambda i: (i, 0)),
            pl.BlockSpec((_R1VM, H), lambda i: (i + nb, 0)),
            pl.BlockSpec((_R1VM, H), lambda i: (i + 2 * nb, 0)),
        ],
        out_specs=pl.BlockSpec((_R1VM, H), _ROW),
        out_shape=jax.ShapeDtypeStruct((N, H), jnp.float32),
    )(vr, vr, vr)


_CHUNK = 128
_NSUB = 16
_CW = 3 * H
_APW = N // _NSUB
_NCHUNK = _APW // _CHUNK
_FPT = NF // _NSUB


_CH2 = 2 * _CHUNK
_NCH2 = _APW // _CH2


def _s2_body(xall, ids2d, acc_out, rows0, rows1, idxb, zb, acc,
             semg0, semg1, sems0, sems1):
    c = lax.axis_index("c")
    s = lax.axis_index("s")
    base = s * _APW
    col = c * H

    zvec = jnp.zeros((16,), jnp.float32)

    def zb_loop(t, _):
        zb[t // 8, pl.ds((t % 8) * 16, 16)] = zvec
        return 0
    lax.fori_loop(0, 16 * 8, zb_loop, 0)

    zd = [pltpu.async_copy(zb, acc.at[pl.ds(s * _FPT + k * 16, 16)], semg0)
          for k in range(_FPT // 16)]
    for d in zd:
        d.wait()

    plsc.subcore_barrier()

    pltpu.sync_copy(ids2d.at[pl.ds(s * _NCHUNK, _NCHUNK)], idxb)

    bufs = [rows0, rows1]
    semg = [semg0, semg1]
    sems = [sems0, sems1]

    def gather(k, b):
        return pltpu.async_copy(
            xall.at[pl.ds(base + k * _CH2, _CH2), pl.ds(col, H)],
            bufs[b], semg[b])

    gd = {0: gather(0, 0)}
    sd = {}
    for k in range(_NCH2):
        b = k % 2
        gd[k].wait()
        if k + 1 < _NCH2:
            if k >= 1:
                sd[k - 1][0].wait()
                sd[k - 1][1].wait()
            gd[k + 1] = gather(k + 1, (k + 1) % 2)
        s1 = pltpu.async_copy(bufs[b].at[pl.ds(0, _CHUNK)],
                              acc.at[idxb.at[2 * k]], sems[b], add=True)
        s2 = pltpu.async_copy(bufs[b].at[pl.ds(_CHUNK, _CHUNK)],
                              acc.at[idxb.at[2 * k + 1]], sems[b], add=True)
        sd[k] = (s1, s2)
    for k in (_NCH2 - 2, _NCH2 - 1):
        sd[k][0].wait()
        sd[k][1].wait()

    plsc.subcore_barrier()

    pltpu.sync_copy(acc.at[pl.ds(s * _FPT, _FPT)],
                    acc_out.at[pl.ds(c * NF + s * _FPT, _FPT)])


_APWH = N // (2 * _NSUB)
_NCH2H = _APWH // _CH2


def _s2h_body(xg, ids2d, acc_out, rows0, rows1, idxb, zb, acc,
              semg0, semg1, sems0, sems1):
    c = lax.axis_index("c")
    s = lax.axis_index("s")
    wid = c * _NSUB + s
    base = wid * _APWH

    zvec = jnp.zeros((16,), jnp.float32)

    def zb_loop(t, _):
        zb[t // 8, pl.ds((t % 8) * 16, 16)] = zvec
        return 0
    lax.fori_loop(0, 16 * 8, zb_loop, 0)

    zd = [pltpu.async_copy(zb, acc.at[pl.ds(s * _FPT + k * 16, 16)], semg0)
          for k in range(_FPT // 16)]
    for d in zd:
        d.wait()

    plsc.subcore_barrier()

    pltpu.sync_copy(ids2d.at[pl.ds(wid * (_APWH // _CHUNK), _APWH // _CHUNK)],
                    idxb)

    bufs = [rows0, rows1]
    semg = [semg0, semg1]
    sems = [sems0, sems1]

    def gather(k, b):
        return pltpu.async_copy(
            xg.at[pl.ds(base + k * _CH2, _CH2)], bufs[b], semg[b])

    gd = {0: gather(0, 0)}
    sd = {}
    for k in range(_NCH2H):
        b = k % 2
        gd[k].wait()
        if k + 1 < _NCH2H:
            if k >= 1:
                sd[k - 1][0].wait()
                sd[k - 1][1].wait()
            gd[k + 1] = gather(k + 1, (k + 1) % 2)
        s1 = pltpu.async_copy(bufs[b].at[pl.ds(0, _CHUNK)],
                              acc.at[idxb.at[2 * k]], sems[b], add=True)
        s2 = pltpu.async_copy(bufs[b].at[pl.ds(_CHUNK, _CHUNK)],
                              acc.at[idxb.at[2 * k + 1]], sems[b], add=True)
        sd[k] = (s1, s2)
    for k in (_NCH2H - 2, _NCH2H - 1):
        sd[k][0].wait()
        sd[k][1].wait()

    plsc.subcore_barrier()

    pltpu.sync_copy(acc.at[pl.ds(s * _FPT, _FPT)],
                    acc_out.at[pl.ds(c * NF + s * _FPT, _FPT)])


def _seg_sum_half_call(xg, ids2d):
    mesh = plsc.VectorSubcoreMesh(core_axis_name="c", subcore_axis_name="s")
    k = pl.kernel(
        _s2h_body,
        out_type=jax.ShapeDtypeStruct((2 * NF, H), jnp.float32),
        mesh=mesh,
        scratch_types=[
            pltpu.VMEM((_CH2, H), jnp.float32),
            pltpu.VMEM((_CH2, H), jnp.float32),
            pltpu.VMEM((_APWH // _CHUNK, _CHUNK), jnp.int32),
            pltpu.VMEM((16, H), jnp.float32),
            pltpu.VMEM_SHARED((NF, H), jnp.float32),
            pltpu.SemaphoreType.DMA,
            pltpu.SemaphoreType.DMA,
            pltpu.SemaphoreType.DMA,
            pltpu.SemaphoreType.DMA,
        ],
    )
    return k(xg, ids2d)


def _seg_sum_call(x2g, ids2d):
    mesh = plsc.VectorSubcoreMesh(core_axis_name="c", subcore_axis_name="s")
    k = pl.kernel(
        _s2_body,
        out_type=jax.ShapeDtypeStruct((2 * NF, H), jnp.float32),
        mesh=mesh,
        scratch_types=[
            pltpu.VMEM((_CH2, H), jnp.float32),
            pltpu.VMEM((_CH2, H), jnp.float32),
            pltpu.VMEM((_NCHUNK, _CHUNK), jnp.int32),
            pltpu.VMEM((16, H), jnp.float32),
            pltpu.VMEM_SHARED((NF, H), jnp.float32),
            pltpu.SemaphoreType.DMA,
            pltpu.SemaphoreType.DMA,
            pltpu.SemaphoreType.DMA,
            pltpu.SemaphoreType.DMA,
        ],
    )
    return k(x2g, ids2d)


_R3 = NF
_NB3 = NF // _R3


def _s3a_body(bfn, by, s0, s1, out):
    small = s0[...] + s1[...]
    B = bfn[...]
    A = by[...]
    cnt = small[:, 0]
    D = small[:, 1]
    E = small[:, 2]
    w = (cnt >= MIN_FRAG).astype(jnp.float32)
    nA = jnp.sqrt(jnp.sum(A * A, axis=1))
    dotAB = jnp.sum(A * B, axis=1)
    intra_num = jnp.sum(w * (cnt - dotAB / jnp.maximum(nA, 1e-12)))
    mnD = D / jnp.maximum(jnp.abs(D), 1e-12)
    vec_num = jnp.sum(w * (cnt - mnD * E))
    denom = jnp.sum(w * cnt)
    idx = lax.broadcasted_iota(jnp.int32, (1, H), 1)
    out[...] = jnp.where(
        idx == 0, intra_num,
        jnp.where(idx == 1, vec_num, jnp.where(idx == 2, denom, 0.0)))


def _s3b_body(byl0, byl1, part, out):
    C = byl0[...] + byl1[...]
    p = part[...]
    nC = jnp.sqrt(jnp.sum(C * C, axis=1, keepdims=True))
    ffn = C / jnp.maximum(nC, 1e-12)
    vpart = jnp.sum(ffn, axis=0)
    sqpart = jnp.sum(ffn * ffn)
    vv = jnp.sum(vpart * vpart)
    d = jnp.maximum(p[0, 2], 1.0)
    intra = p[0, 0] / d
    vec = p[0, 1] / d
    inter = (vv - sqpart) / (float(NF) * NF - NF + 1e-6)
    total = intra + 0.05 * vec + 0.2 * inter
    out[...] = jnp.reshape(0.03 * total * 0.05, (1, 1))


def _stage3a(acc_fn_y, acc_sm):
    return pl.pallas_call(
        _s3a_body,
        grid=(1,),
        in_specs=[
            pl.BlockSpec((NF, H), lambda i: (0, 0)),
            pl.BlockSpec((NF, H), lambda i: (1, 0)),
            pl.BlockSpec((NF, H), lambda i: (0, 0)),
            pl.BlockSpec((NF, H), lambda i: (1, 0)),
        ],
        out_specs=pl.BlockSpec((1, H), lambda i: (0, 0)),
        out_shape=jax.ShapeDtypeStruct((1, H), jnp.float32),
    )(acc_fn_y, acc_fn_y, acc_sm, acc_sm)


def _stage3b(acc_yl, part):
    return pl.pallas_call(
        _s3b_body,
        grid=(1,),
        in_specs=[
            pl.BlockSpec((NF, H), lambda i: (0, 0)),
            pl.BlockSpec((NF, H), lambda i: (1, 0)),
            pl.BlockSpec((1, H), lambda i: (0, 0)),
        ],
        out_specs=pl.BlockSpec((1, 1), lambda i: (0, 0)),
        out_shape=jax.ShapeDtypeStruct((1, 1), jnp.float32),
    )(acc_yl, acc_yl, part)


@jax.jit
def kernel(scalar_short, scalar_long, vector_short, vector_long, fragment_ids,
           ln_s_w, ln_s_b, lin_s_W, lin_s_b, ln_l_w, ln_l_b, lin_l_W, lin_l_b):
    vt = jnp.transpose(vector_short, (1, 0, 2))
    ids2d = jnp.reshape(fragment_ids.astype(jnp.int32), (N // _CHUNK, _CHUNK))
    xa = _stage1a(scalar_short,
                  jnp.reshape(ln_s_w, (1, H)), jnp.reshape(ln_s_b, (1, H)),
                  lin_s_W, jnp.reshape(lin_s_b, (1, H)))
    acc_fn_y = _seg_sum_call(xa, ids2d)
    xvm = _stage1vm(vt)
    acc_sm = _seg_sum_half_call(xvm, ids2d)
    xyl = _stage1yl(scalar_long,
                    jnp.reshape(ln_l_w, (1, H)), jnp.reshape(ln_l_b, (1, H)),
                    lin_l_W, jnp.reshape(lin_l_b, (1, H)))
    acc_yl = _seg_sum_half_call(xyl, ids2d)
    part = _stage3a(acc_fn_y, acc_sm)
    out = _stage3b(acc_yl, part)
    return jnp.reshape(out, ())


# speedup vs baseline: 1.0347x; 1.0347x over previous
import jax
import jax.numpy as jnp
from jax import lax
from jax.experimental import pallas as pl
from jax.experimental.pallas import tpu as pltpu
from jax.experimental.pallas import tpu_sc as plsc

N = 32768
H = 128
NF = 4096
MIN_FRAG = 3.0

_R1 = 4096
_R1VM = 4096


def _proj(x, lw, lb, W, b):
    m = jnp.mean(x, axis=1, keepdims=True)
    xc = x - m
    v = jnp.mean(xc * xc, axis=1, keepdims=True)
    h = xc * lax.rsqrt(v + 1e-5) * lw + lb
    y = lax.dot_general(h, W, (((1,), (1,)), ((), ())),
                        preferred_element_type=jnp.float32) + b
    return y * jax.nn.sigmoid(y)


def _s1a_body(ss, lnsw, lnsb, Ws, bs, xa):
    ys = _proj(ss[...], lnsw[...], lnsb[...], Ws[...], bs[...])
    rs = jnp.sqrt(jnp.sum(ys * ys, axis=1, keepdims=True))
    fn = ys / jnp.maximum(rs, 1e-12)
    xa[...] = jnp.concatenate([fn, ys], axis=1)


def _s1yl_body(sl, lnlw, lnlb, Wl, bl, xb):
    xb[...] = _proj(sl[...], lnlw[...], lnlb[...], Wl[...], bl[...])


def _s1vm_body(v0, v1, v2, xb):
    a = v0[...]
    b = v1[...]
    c = v2[...]
    vm = ((jnp.sqrt(jnp.sum(a * a, axis=1)) + jnp.sqrt(jnp.sum(b * b, axis=1))
           + jnp.sqrt(jnp.sum(c * c, axis=1))) * (1.0 / 3.0))[:, None]
    vmn = vm / jnp.maximum(vm, 1e-12)
    ones = jnp.ones_like(vm)
    zeros = jnp.zeros((vm.shape[0], H - 3), jnp.float32)
    xb[...] = jnp.concatenate([ones, vm, vmn, zeros], axis=1)


_ROW = lambda i: (i, 0)
_FULL = lambda i: (0, 0)


def _stage1a(ss, lnsw, lnsb, Ws, bs):
    return pl.pallas_call(
        _s1a_body,
        grid=(N // _R1,),
        in_specs=[
            pl.BlockSpec((_R1, H), _ROW),
            pl.BlockSpec((1, H), _FULL),
            pl.BlockSpec((1, H), _FULL),
            pl.BlockSpec((H, H), _FULL),
            pl.BlockSpec((1, H), _FULL),
        ],
        out_specs=pl.BlockSpec((_R1, 2 * H), _ROW),
        out_shape=jax.ShapeDtypeStruct((N, 2 * H), jnp.float32),
    )(ss, lnsw, lnsb, Ws, bs)


def _stage1yl(sl, lnlw, lnlb, Wl, bl):
    return pl.pallas_call(
        _s1yl_body,
        grid=(N // _R1,),
        in_specs=[
            pl.BlockSpec((_R1, H), _ROW),
            pl.BlockSpec((1, H), _FULL),
            pl.BlockSpec((1, H), _FULL),
            pl.BlockSpec((H, H), _FULL),
            pl.BlockSpec((1, H), _FULL),
        ],
        out_specs=pl.BlockSpec((_R1, H), _ROW),
        out_shape=jax.ShapeDtypeStruct((N, H), jnp.float32),
    )(sl, lnlw, lnlb, Wl, bl)


def _stage1vm(vs):
    nb = N // _R1VM
    vr = jnp.reshape(vs, (3 * N, H))
    return pl.pallas_call(
        _s1vm_body,
        grid=(nb,),
        in_specs=[
            pl.BlockSpec((_R1VM, H), lambda i: (i, 0)),
            pl.BlockSpec((_R1VM, H), lambda i: (i + nb, 0)),
            pl.BlockSpec((_R1VM, H), lambda i: (i + 2 * nb, 0)),
        ],
        out_specs=pl.BlockSpec((_R1VM, H), _ROW),
        out_shape=jax.ShapeDtypeStruct((N, H), jnp.float32),
    )(vr, vr, vr)


_CHUNK = 128
_NSUB = 16
_CW = 3 * H
_APW = N // _NSUB
_NCHUNK = _APW // _CHUNK
_FPT = NF // _NSUB


_CH2 = 2 * _CHUNK
_NCH2 = _APW // _CH2


def _s2_body(xall, ids2d, acc_out, rows0, rows1, idxb, zb, acc,
             semg0, semg1, sems0, sems1):
    c = lax.axis_index("c")
    s = lax.axis_index("s")
    base = s * _APW
    col = c * H

    zvec = jnp.zeros((16,), jnp.float32)

    def zb_loop(t, _):
        zb[t // 8, pl.ds((t % 8) * 16, 16)] = zvec
        return 0
    lax.fori_loop(0, 16 * 8, zb_loop, 0)

    bufs = [rows0, rows1]
    semg = [semg0, semg1]
    sems = [sems0, sems1]

    def gather(k, b):
        return pltpu.async_copy(
            xall.at[pl.ds(base + k * _CH2, _CH2), pl.ds(col, H)],
            bufs[b], semg[b])

    zd = [pltpu.async_copy(zb, acc.at[pl.ds(s * _FPT + k * 16, 16)], sems0)
          for k in range(_FPT // 16)]
    idxd = pltpu.async_copy(ids2d.at[pl.ds(s * _NCHUNK, _NCHUNK)], idxb,
                            sems1)
    gd = {0: gather(0, 0), 1: gather(1, 1)}
    for d in zd:
        d.wait()
    idxd.wait()

    plsc.subcore_barrier()

    sd = {}
    for k in range(_NCH2):
        b = k % 2
        gd[k].wait()
        s1 = pltpu.async_copy(bufs[b].at[pl.ds(0, _CHUNK)],
                              acc.at[idxb.at[2 * k]], sems[b], add=True)
        s2 = pltpu.async_copy(bufs[b].at[pl.ds(_CHUNK, _CHUNK)],
                              acc.at[idxb.at[2 * k + 1]], sems[b], add=True)
        sd[k] = (s1, s2)
        if k + 2 < _NCH2:
            s1.wait()
            s2.wait()
            gd[k + 2] = gather(k + 2, b)
    for k in (_NCH2 - 2, _NCH2 - 1):
        sd[k][0].wait()
        sd[k][1].wait()

    plsc.subcore_barrier()

    pltpu.sync_copy(acc.at[pl.ds(s * _FPT, _FPT)],
                    acc_out.at[pl.ds(c * NF + s * _FPT, _FPT)])


_APWH = N // (2 * _NSUB)
_NCH2H = _APWH // _CH2


def _s2h_body(xg, ids2d, acc_out, rows0, rows1, idxb, zb, acc,
              semg0, semg1, sems0, sems1):
    c = lax.axis_index("c")
    s = lax.axis_index("s")
    wid = c * _NSUB + s
    base = wid * _APWH

    zvec = jnp.zeros((16,), jnp.float32)

    def zb_loop(t, _):
        zb[t // 8, pl.ds((t % 8) * 16, 16)] = zvec
        return 0
    lax.fori_loop(0, 16 * 8, zb_loop, 0)

    bufs = [rows0, rows1]
    semg = [semg0, semg1]
    sems = [sems0, sems1]

    def gather(k, b):
        return pltpu.async_copy(
            xg.at[pl.ds(base + k * _CH2, _CH2)], bufs[b], semg[b])

    zd = [pltpu.async_copy(zb, acc.at[pl.ds(s * _FPT + k * 16, 16)], sems0)
          for k in range(_FPT // 16)]
    idxd = pltpu.async_copy(
        ids2d.at[pl.ds(wid * (_APWH // _CHUNK), _APWH // _CHUNK)], idxb,
        sems1)
    gd = {0: gather(0, 0), 1: gather(1, 1)}
    for d in zd:
        d.wait()
    idxd.wait()

    plsc.subcore_barrier()

    sd = {}
    for k in range(_NCH2H):
        b = k % 2
        gd[k].wait()
        s1 = pltpu.async_copy(bufs[b].at[pl.ds(0, _CHUNK)],
                              acc.at[idxb.at[2 * k]], sems[b], add=True)
        s2 = pltpu.async_copy(bufs[b].at[pl.ds(_CHUNK, _CHUNK)],
                              acc.at[idxb.at[2 * k + 1]], sems[b], add=True)
        sd[k] = (s1, s2)
        if k + 2 < _NCH2H:
            s1.wait()
            s2.wait()
            gd[k + 2] = gather(k + 2, b)
    for k in (_NCH2H - 2, _NCH2H - 1):
        sd[k][0].wait()
        sd[k][1].wait()

    plsc.subcore_barrier()

    pltpu.sync_copy(acc.at[pl.ds(s * _FPT, _FPT)],
                    acc_out.at[pl.ds(c * NF + s * _FPT, _FPT)])


def _seg_sum_half_call(xg, ids2d):
    mesh = plsc.VectorSubcoreMesh(core_axis_name="c", subcore_axis_name="s")
    k = pl.kernel(
        _s2h_body,
        out_type=jax.ShapeDtypeStruct((2 * NF, H), jnp.float32),
        mesh=mesh,
        scratch_types=[
            pltpu.VMEM((_CH2, H), jnp.float32),
            pltpu.VMEM((_CH2, H), jnp.float32),
            pltpu.VMEM((_APWH // _CHUNK, _CHUNK), jnp.int32),
            pltpu.VMEM((16, H), jnp.float32),
            pltpu.VMEM_SHARED((NF, H), jnp.float32),
            pltpu.SemaphoreType.DMA,
            pltpu.SemaphoreType.DMA,
            pltpu.SemaphoreType.DMA,
            pltpu.SemaphoreType.DMA,
        ],
    )
    return k(xg, ids2d)


def _seg_sum_call(x2g, ids2d):
    mesh = plsc.VectorSubcoreMesh(core_axis_name="c", subcore_axis_name="s")
    k = pl.kernel(
        _s2_body,
        out_type=jax.ShapeDtypeStruct((2 * NF, H), jnp.float32),
        mesh=mesh,
        scratch_types=[
            pltpu.VMEM((_CH2, H), jnp.float32),
            pltpu.VMEM((_CH2, H), jnp.float32),
            pltpu.VMEM((_NCHUNK, _CHUNK), jnp.int32),
            pltpu.VMEM((16, H), jnp.float32),
            pltpu.VMEM_SHARED((NF, H), jnp.float32),
            pltpu.SemaphoreType.DMA,
            pltpu.SemaphoreType.DMA,
            pltpu.SemaphoreType.DMA,
            pltpu.SemaphoreType.DMA,
        ],
    )
    return k(x2g, ids2d)


_R3 = NF
_NB3 = NF // _R3


def _s3a_body(bfn, by, s0, s1, out):
    small = s0[...] + s1[...]
    B = bfn[...]
    A = by[...]
    cnt = small[:, 0]
    D = small[:, 1]
    E = small[:, 2]
    w = (cnt >= MIN_FRAG).astype(jnp.float32)
    nA = jnp.sqrt(jnp.sum(A * A, axis=1))
    dotAB = jnp.sum(A * B, axis=1)
    intra_num = jnp.sum(w * (cnt - dotAB / jnp.maximum(nA, 1e-12)))
    mnD = D / jnp.maximum(jnp.abs(D), 1e-12)
    vec_num = jnp.sum(w * (cnt - mnD * E))
    denom = jnp.sum(w * cnt)
    idx = lax.broadcasted_iota(jnp.int32, (1, H), 1)
    out[...] = jnp.where(
        idx == 0, intra_num,
        jnp.where(idx == 1, vec_num, jnp.where(idx == 2, denom, 0.0)))


def _s3b_body(byl0, byl1, part, out):
    C = byl0[...] + byl1[...]
    p = part[...]
    nC = jnp.sqrt(jnp.sum(C * C, axis=1, keepdims=True))
    ffn = C / jnp.maximum(nC, 1e-12)
    vpart = jnp.sum(ffn, axis=0)
    sqpart = jnp.sum(ffn * ffn)
    vv = jnp.sum(vpart * vpart)
    d = jnp.maximum(p[0, 2], 1.0)
    intra = p[0, 0] / d
    vec = p[0, 1] / d
    inter = (vv - sqpart) / (float(NF) * NF - NF + 1e-6)
    total = intra + 0.05 * vec + 0.2 * inter
    out[...] = jnp.reshape(0.03 * total * 0.05, (1, 1))


def _stage3a(acc_fn_y, acc_sm):
    return pl.pallas_call(
        _s3a_body,
        grid=(1,),
        in_specs=[
            pl.BlockSpec((NF, H), lambda i: (0, 0)),
            pl.BlockSpec((NF, H), lambda i: (1, 0)),
            pl.BlockSpec((NF, H), lambda i: (0, 0)),
            pl.BlockSpec((NF, H), lambda i: (1, 0)),
        ],
        out_specs=pl.BlockSpec((1, H), lambda i: (0, 0)),
        out_shape=jax.ShapeDtypeStruct((1, H), jnp.float32),
    )(acc_fn_y, acc_fn_y, acc_sm, acc_sm)


def _stage3b(acc_yl, part):
    return pl.pallas_call(
        _s3b_body,
        grid=(1,),
        in_specs=[
            pl.BlockSpec((NF, H), lambda i: (0, 0)),
            pl.BlockSpec((NF, H), lambda i: (1, 0)),
            pl.BlockSpec((1, H), lambda i: (0, 0)),
        ],
        out_specs=pl.BlockSpec((1, 1), lambda i: (0, 0)),
        out_shape=jax.ShapeDtypeStruct((1, 1), jnp.float32),
    )(acc_yl, acc_yl, part)


@jax.jit
def kernel(scalar_short, scalar_long, vector_short, vector_long, fragment_ids,
           ln_s_w, ln_s_b, lin_s_W, lin_s_b, ln_l_w, ln_l_b, lin_l_W, lin_l_b):
    vt = jnp.transpose(vector_short, (1, 0, 2))
    ids2d = jnp.reshape(fragment_ids.astype(jnp.int32), (N // _CHUNK, _CHUNK))
    xa = _stage1a(scalar_short,
                  jnp.reshape(ln_s_w, (1, H)), jnp.reshape(ln_s_b, (1, H)),
                  lin_s_W, jnp.reshape(lin_s_b, (1, H)))
    acc_fn_y = _seg_sum_call(xa, ids2d)
    xvm = _stage1vm(vt)
    acc_sm = _seg_sum_half_call(xvm, ids2d)
    xyl = _stage1yl(scalar_long,
                    jnp.reshape(ln_l_w, (1, H)), jnp.reshape(ln_l_b, (1, H)),
                    lin_l_W, jnp.reshape(lin_l_b, (1, H)))
    acc_yl = _seg_sum_half_call(xyl, ids2d)
    part = _stage3a(acc_fn_y, acc_sm)
    out = _stage3b(acc_yl, part)
    return jnp.reshape(out, ())
